# initial kernel scaffold (unmeasured)
import jax
import jax.numpy as jnp
from jax import lax
from jax.experimental import pallas as pl
from jax.experimental.pallas import tpu as pltpu


def _exchange(x, d2):

    def body(x_ref, d_ref, ox_ref, od_ref, sems):
        my_x = lax.axis_index("x")
        my_y = lax.axis_index("y")
        nbr = (my_x, 1 - my_y)

        barrier = pltpu.get_barrier_semaphore()
        pl.semaphore_signal(
            barrier, inc=1, device_id=nbr, device_id_type=pl.DeviceIdType.MESH
        )
        pl.semaphore_wait(barrier, 1)

        cx = pltpu.make_async_remote_copy(
            src_ref=x_ref,
            dst_ref=ox_ref,
            send_sem=sems.at[0],
            recv_sem=sems.at[1],
            device_id=nbr,
            device_id_type=pl.DeviceIdType.MESH,
        )
        cd = pltpu.make_async_remote_copy(
            src_ref=d_ref,
            dst_ref=od_ref,
            send_sem=sems.at[2],
            recv_sem=sems.at[3],
            device_id=nbr,
            device_id_type=pl.DeviceIdType.MESH,
        )
        cx.start()
        cd.start()
        cx.wait()
        cd.wait()

    return pl.pallas_call(
        body,
        out_shape=(
            jax.ShapeDtypeStruct(x.shape, x.dtype),
            jax.ShapeDtypeStruct(d2.shape, d2.dtype),
        ),
        in_specs=[
            pl.BlockSpec(memory_space=pltpu.ANY),
            pl.BlockSpec(memory_space=pltpu.ANY),
        ],
        out_specs=(
            pl.BlockSpec(memory_space=pltpu.ANY),
            pl.BlockSpec(memory_space=pltpu.ANY),
        ),
        scratch_shapes=[pltpu.SemaphoreType.DMA((4,))],
        compiler_params=pltpu.CompilerParams(collective_id=0),
    )(x, d2)


def kernel(x, dest):
    n, _ = x.shape
    d2 = dest.reshape(8, n // 8)
    ox, od2 = _exchange(x, d2)
    od = od2.reshape(n)

    r = lax.axis_index("y")
    dest_all = jnp.where(
        r == 0, jnp.concatenate([dest, od]), jnp.concatenate([od, dest])
    )
    x_all = jnp.where(
        r == 0,
        jnp.concatenate([x, ox], axis=0),
        jnp.concatenate([ox, x], axis=0),
    )
    order = jnp.argsort((dest_all != r).astype(jnp.int32), stable=True)
    return x_all[order[:n]]


# baseline (device time: 490670 ns/iter reference)
import jax
import jax.numpy as jnp
from jax import lax
from jax.experimental import pallas as pl
from jax.experimental.pallas import tpu as pltpu


def _exchange(x, d2):

    def body(x_ref, d_ref, ox_ref, od_ref, sems):
        my_x = lax.axis_index("x")
        my_y = lax.axis_index("y")
        nbr = (my_x, 1 - my_y)

        barrier = pltpu.get_barrier_semaphore()
        pl.semaphore_signal(
            barrier, inc=1, device_id=nbr, device_id_type=pl.DeviceIdType.MESH
        )
        pl.semaphore_wait(barrier, 1)

        cx = pltpu.make_async_remote_copy(
            src_ref=x_ref,
            dst_ref=ox_ref,
            send_sem=sems.at[0],
            recv_sem=sems.at[1],
            device_id=nbr,
            device_id_type=pl.DeviceIdType.MESH,
        )
        cd = pltpu.make_async_remote_copy(
            src_ref=d_ref,
            dst_ref=od_ref,
            send_sem=sems.at[2],
            recv_sem=sems.at[3],
            device_id=nbr,
            device_id_type=pl.DeviceIdType.MESH,
        )
        cx.start()
        cd.start()
        cx.wait()
        cd.wait()

    return pl.pallas_call(
        body,
        out_shape=(
            jax.ShapeDtypeStruct(x.shape, x.dtype),
            jax.ShapeDtypeStruct(d2.shape, d2.dtype),
        ),
        in_specs=[
            pl.BlockSpec(memory_space=pl.ANY),
            pl.BlockSpec(memory_space=pl.ANY),
        ],
        out_specs=(
            pl.BlockSpec(memory_space=pl.ANY),
            pl.BlockSpec(memory_space=pl.ANY),
        ),
        scratch_shapes=[pltpu.SemaphoreType.DMA((4,))],
        compiler_params=pltpu.CompilerParams(collective_id=0),
    )(x, d2)


def kernel(x, dest):
    n, _ = x.shape
    d2 = dest.reshape(8, n // 8)
    ox, od2 = _exchange(x, d2)
    od = od2.reshape(n)

    r = lax.axis_index("y")
    dest_all = jnp.where(
        r == 0, jnp.concatenate([dest, od]), jnp.concatenate([od, dest])
    )
    x_all = jnp.where(
        r == 0,
        jnp.concatenate([x, ox], axis=0),
        jnp.concatenate([ox, x], axis=0),
    )
    order = jnp.argsort((dest_all != r).astype(jnp.int32), stable=True)
    return x_all[order[:n]]


# device time: 201623 ns/iter; 2.4336x vs baseline; 2.4336x over previous
import jax
import jax.numpy as jnp
from jax import lax
from jax.experimental import pallas as pl
from jax.experimental.pallas import tpu as pltpu

N = 4096
CH = 256
MAX_CHUNKS = N // CH


def _exchange_dest(d2):

    def body(d_ref, od_ref, sems):
        my_x = lax.axis_index("x")
        my_y = lax.axis_index("y")
        nbr = (my_x, 1 - my_y)

        barrier = pltpu.get_barrier_semaphore()
        pl.semaphore_signal(
            barrier, inc=1, device_id=nbr, device_id_type=pl.DeviceIdType.MESH
        )
        pl.semaphore_wait(barrier, 1)

        cd = pltpu.make_async_remote_copy(
            src_ref=d_ref,
            dst_ref=od_ref,
            send_sem=sems.at[0],
            recv_sem=sems.at[1],
            device_id=nbr,
            device_id_type=pl.DeviceIdType.MESH,
        )
        cd.start()
        cd.wait()

    return pl.pallas_call(
        body,
        out_shape=jax.ShapeDtypeStruct(d2.shape, d2.dtype),
        in_specs=[pl.BlockSpec(memory_space=pl.ANY)],
        out_specs=pl.BlockSpec(memory_space=pl.ANY),
        scratch_shapes=[pltpu.SemaphoreType.DMA((2,))],
        compiler_params=pltpu.CompilerParams(collective_id=1),
    )(d2)


def _a2av(x3, keep_pos, send_pos, meta):

    def body(x_ref, kp_ref, sp_ref, meta_ref, out_ref, sbuf_ref, ssems, rsems):
        my_x = lax.axis_index("x")
        my_y = lax.axis_index("y")
        nbr = (my_x, 1 - my_y)

        k_send = meta_ref[0]
        k_recv = meta_ref[1]
        dst_off = meta_ref[2]
        recv_off = meta_ref[3]
        n_sc = meta_ref[4]
        n_rc = meta_ref[5]

        barrier = pltpu.get_barrier_semaphore()
        pl.semaphore_signal(
            barrier, inc=1, device_id=nbr, device_id_type=pl.DeviceIdType.MESH
        )
        pl.semaphore_wait(barrier, 1)

        def row_body(i, carry):
            row = x_ref[pl.ds(i, 1)]
            kp = kp_ref[i]
            sp = sp_ref[i]

            @pl.when(kp >= 0)
            def _():
                out_ref[pl.ds(kp, 1)] = row

            @pl.when(sp >= 0)
            def _():
                sbuf_ref[pl.ds(sp, 1)] = row

            return carry

        lax.fori_loop(0, N, row_body, 0)

        for c in range(MAX_CHUNKS):
            @pl.when(c < n_sc)
            def _():
                s = jnp.minimum(c * CH, k_send - CH)
                rdma = pltpu.make_async_remote_copy(
                    src_ref=sbuf_ref.at[pl.ds(s, CH)],
                    dst_ref=out_ref.at[pl.ds(dst_off + s, CH)],
                    send_sem=ssems.at[c],
                    recv_sem=rsems.at[c],
                    device_id=nbr,
                    device_id_type=pl.DeviceIdType.MESH,
                )
                rdma.start()

        for c in range(MAX_CHUNKS):
            @pl.when(c < n_sc)
            def _():
                pltpu.make_async_remote_copy(
                    src_ref=sbuf_ref.at[pl.ds(0, CH)],
                    dst_ref=out_ref.at[pl.ds(0, CH)],
                    send_sem=ssems.at[c],
                    recv_sem=rsems.at[c],
                    device_id=nbr,
                    device_id_type=pl.DeviceIdType.MESH,
                ).wait_send()

            @pl.when(c < n_rc)
            def _():
                rs = recv_off + jnp.minimum(c * CH, k_recv - CH)
                pltpu.make_async_remote_copy(
                    src_ref=sbuf_ref.at[pl.ds(0, CH)],
                    dst_ref=out_ref.at[pl.ds(rs, CH)],
                    send_sem=ssems.at[c],
                    recv_sem=rsems.at[c],
                    device_id=nbr,
                    device_id_type=pl.DeviceIdType.MESH,
                ).wait_recv()

    return pl.pallas_call(
        body,
        out_shape=jax.ShapeDtypeStruct((N, 8, 128), x3.dtype),
        in_specs=[
            pl.BlockSpec(memory_space=pltpu.VMEM),
            pl.BlockSpec(memory_space=pltpu.SMEM),
            pl.BlockSpec(memory_space=pltpu.SMEM),
            pl.BlockSpec(memory_space=pltpu.SMEM),
        ],
        out_specs=pl.BlockSpec(memory_space=pltpu.VMEM),
        scratch_shapes=[
            pltpu.VMEM((N, 8, 128), x3.dtype),
            pltpu.SemaphoreType.DMA((MAX_CHUNKS,)),
            pltpu.SemaphoreType.DMA((MAX_CHUNKS,)),
        ],
        compiler_params=pltpu.CompilerParams(collective_id=0),
    )(x3, keep_pos, send_pos, meta)


def kernel(x, dest):
    r = lax.axis_index("y")
    od = _exchange_dest(dest.reshape(8, N // 8)).reshape(N)

    m_keep = (dest == r).astype(jnp.int32)
    m_send = 1 - m_keep
    cs_keep = jnp.cumsum(m_keep)
    cs_send = jnp.cumsum(m_send)
    k_keep = cs_keep[-1]
    k_send = N - k_keep
    k_recv = jnp.sum((od == r).astype(jnp.int32))

    own_off = jnp.where(r == 0, 0, k_recv)
    recv_off = jnp.where(r == 0, k_keep, 0)
    dst_off = jnp.where(r == 0, 0, jnp.sum((od == 1 - r).astype(jnp.int32)))

    keep_pos = jnp.where(m_keep == 1, own_off + cs_keep - 1, -1).astype(jnp.int32)
    send_pos = jnp.where(m_send == 1, cs_send - 1, -1).astype(jnp.int32)

    n_sc = (k_send + CH - 1) // CH
    n_rc = (k_recv + CH - 1) // CH
    meta = jnp.stack(
        [k_send, k_recv, dst_off, recv_off, n_sc, n_rc, 0, 0]
    ).astype(jnp.int32)

    out3 = _a2av(x.reshape(N, 8, 128), keep_pos, send_pos, meta)
    return out3.reshape(N, 1024)
